# baseline (device time: 73433 ns/iter reference)
import functools

import jax
import jax.numpy as jnp
from jax import lax
from jax.experimental import pallas as pl
from jax.experimental.pallas import tpu as pltpu

N_DEV = 8
M = 1024
N = 1024
CHUNK = M // N_DEV
N_HOPS = 2 * (N_DEV - 1)


def kernel(x, w_mat):
    m, k = x.shape
    _, n = w_mat.shape

    def body(x_ref, w_ref, out_ref, acc_ref, send_ref, recv_ref,
             send_sems, recv_sems):
        d = lax.axis_index("i")
        right = (d + 1) % N_DEV
        left = (d + N_DEV - 1) % N_DEV

        barrier_sem = pltpu.get_barrier_semaphore()
        for nbr in (left, right):
            pl.semaphore_signal(
                barrier_sem, inc=1,
                device_id=(nbr,), device_id_type=pl.DeviceIdType.MESH,
            )
        pl.semaphore_wait(barrier_sem, 2)

        acc_ref[...] = jnp.dot(
            x_ref[...], w_ref[...], preferred_element_type=jnp.float32
        )

        for s in range(N_DEV - 1):
            c_send = (d - s) % N_DEV
            send_ref[s, :, :] = acc_ref[
                pl.ds(c_send * CHUNK, CHUNK), :
            ].astype(jnp.bfloat16)
            rdma = pltpu.make_async_remote_copy(
                src_ref=send_ref.at[s],
                dst_ref=recv_ref.at[s],
                send_sem=send_sems.at[s],
                recv_sem=recv_sems.at[s],
                device_id=(right,),
                device_id_type=pl.DeviceIdType.MESH,
            )
            rdma.start()
            rdma.wait()
            c_recv = (d - 1 - s) % N_DEV
            acc_ref[pl.ds(c_recv * CHUNK, CHUNK), :] = (
                acc_ref[pl.ds(c_recv * CHUNK, CHUNK), :]
                + recv_ref[s, :, :].astype(jnp.float32)
            )

        own = (d + 1) % N_DEV
        z = acc_ref[pl.ds(own * CHUNK, CHUNK), :]
        y = z * jax.nn.sigmoid(z)
        out_ref[pl.ds(own * CHUNK, CHUNK), :] = y
        send_ref[N_DEV - 1, :, :] = y.astype(jnp.bfloat16)

        for t in range(N_DEV - 1):
            hop = (N_DEV - 1) + t
            src = send_ref.at[N_DEV - 1] if t == 0 else recv_ref.at[hop - 1]
            rdma = pltpu.make_async_remote_copy(
                src_ref=src,
                dst_ref=recv_ref.at[hop],
                send_sem=send_sems.at[hop],
                recv_sem=recv_sems.at[hop],
                device_id=(right,),
                device_id_type=pl.DeviceIdType.MESH,
            )
            rdma.start()
            rdma.wait()
            c = (d - t) % N_DEV
            out_ref[pl.ds(c * CHUNK, CHUNK), :] = (
                recv_ref[hop, :, :].astype(jnp.float32)
            )

    return pl.pallas_call(
        body,
        out_shape=jax.ShapeDtypeStruct((M, N), jnp.float32),
        in_specs=[
            pl.BlockSpec(memory_space=pltpu.VMEM),
            pl.BlockSpec(memory_space=pltpu.VMEM),
        ],
        out_specs=pl.BlockSpec(memory_space=pltpu.VMEM),
        scratch_shapes=[
            pltpu.VMEM((M, N), jnp.float32),
            pltpu.VMEM((N_DEV, CHUNK, N), jnp.bfloat16),
            pltpu.VMEM((N_HOPS, CHUNK, N), jnp.bfloat16),
            pltpu.SemaphoreType.DMA((N_HOPS,)),
            pltpu.SemaphoreType.DMA((N_HOPS,)),
        ],
        compiler_params=pltpu.CompilerParams(collective_id=0),
    )(x, w_mat)


# device time: 35029 ns/iter; 2.0963x vs baseline; 2.0963x over previous
import jax
import jax.numpy as jnp
from jax import lax
from jax.experimental import pallas as pl
from jax.experimental.pallas import tpu as pltpu

N_DEV = 8
M = 1024
N = 1024

TREES = ((0, 384), (384, 384), (768, 256))
AXORDER = ((0, 1, 2), (1, 2, 0), (2, 0, 1))
N_EX = 9


def kernel(x, w_mat):
    def body(x_ref, w_ref, out_ref, acc_ref, ag_ref, *rest):
        rs_send = rest[0:N_EX]
        rs_recv = rest[N_EX:2 * N_EX]
        rs_send_sems, rs_recv_sems, ag_send_sems, ag_recv_sems = rest[2 * N_EX:]

        d = lax.axis_index("i")
        m = d % 4
        base = d - m
        partners = (
            base + (m ^ 1),
            base + (m ^ 3),
            d ^ 4,
        )
        bits = (
            (m ^ (m >> 1)) & 1,
            m >> 1,
            d >> 2,
        )

        barrier_sem = pltpu.get_barrier_semaphore()
        for ax in range(3):
            pl.semaphore_signal(
                barrier_sem, inc=1,
                device_id=(partners[ax],),
                device_id_type=pl.DeviceIdType.MESH,
            )
        pl.semaphore_wait(barrier_sem, 3)

        acc_ref[...] = jnp.dot(
            x_ref[...], w_ref[...], preferred_element_type=jnp.float32
        )

        starts = [jnp.int32(r0) for r0, _ in TREES]
        sizes = [r for _, r in TREES]
        for s in range(3):
            rdmas = []
            for t in range(3):
                half = sizes[t] // 2
                ax = AXORDER[t][s]
                b = bits[ax]
                send_start = starts[t] + (1 - b) * half
                i = t * 3 + s
                rs_send[i][...] = acc_ref[
                    pl.ds(send_start, half), :
                ].astype(jnp.bfloat16)
                rdma = pltpu.make_async_remote_copy(
                    src_ref=rs_send[i],
                    dst_ref=rs_recv[i],
                    send_sem=rs_send_sems.at[i],
                    recv_sem=rs_recv_sems.at[i],
                    device_id=(partners[ax],),
                    device_id_type=pl.DeviceIdType.MESH,
                )
                rdma.start()
                rdmas.append(rdma)
                starts[t] = starts[t] + b * half
                sizes[t] = half
            for t in range(3):
                rdmas[t].wait()
            for t in range(3):
                i = t * 3 + s
                acc_ref[pl.ds(starts[t], sizes[t]), :] = (
                    acc_ref[pl.ds(starts[t], sizes[t]), :]
                    + rs_recv[i][...].astype(jnp.float32)
                )

        for t in range(3):
            z = acc_ref[pl.ds(starts[t], sizes[t]), :]
            ag_ref[pl.ds(starts[t], sizes[t]), :] = (
                z * jax.nn.sigmoid(z)
            ).astype(jnp.bfloat16)

        for s in range(3):
            rdmas = []
            for t in range(3):
                ax = AXORDER[t][2 - s]
                b = bits[ax]
                i = t * 3 + s
                rdma = pltpu.make_async_remote_copy(
                    src_ref=ag_ref.at[pl.ds(starts[t], sizes[t])],
                    dst_ref=ag_ref.at[pl.ds(starts[t], sizes[t])],
                    send_sem=ag_send_sems.at[i],
                    recv_sem=ag_recv_sems.at[i],
                    device_id=(partners[ax],),
                    device_id_type=pl.DeviceIdType.MESH,
                )
                rdma.start()
                rdmas.append(rdma)
                starts[t] = starts[t] - b * sizes[t]
                sizes[t] = sizes[t] * 2
            for t in range(3):
                rdmas[t].wait()

        out_ref[...] = ag_ref[...].astype(jnp.float32)

    scratch = [pltpu.VMEM((M, N), jnp.float32), pltpu.VMEM((M, N), jnp.bfloat16)]
    for _, r in TREES:
        for s in range(3):
            scratch.append(pltpu.VMEM((r // 2 ** (s + 1), N), jnp.bfloat16))
    for _, r in TREES:
        for s in range(3):
            scratch.append(pltpu.VMEM((r // 2 ** (s + 1), N), jnp.bfloat16))
    scratch += [pltpu.SemaphoreType.DMA((N_EX,)) for _ in range(4)]

    return pl.pallas_call(
        body,
        out_shape=jax.ShapeDtypeStruct((M, N), jnp.float32),
        in_specs=[
            pl.BlockSpec(memory_space=pltpu.VMEM),
            pl.BlockSpec(memory_space=pltpu.VMEM),
        ],
        out_specs=pl.BlockSpec(memory_space=pltpu.VMEM),
        scratch_shapes=scratch,
        compiler_params=pltpu.CompilerParams(collective_id=0),
    )(x, w_mat)


# device time: 32606 ns/iter; 2.2521x vs baseline; 1.0743x over previous
import jax
import jax.numpy as jnp
from jax import lax
from jax.experimental import pallas as pl
from jax.experimental.pallas import tpu as pltpu

N_DEV = 8
M = 1024
N = 1024

TREES = ((0, 352), (352, 352), (704, 320))
AXORDER = ((0, 1, 2), (1, 2, 0), (2, 0, 1))
N_EX = 12


def kernel(x, w_mat):
    def body(x_ref, w_ref, out_ref, acc_ref, ag_ref, *rest):
        send_bufs = rest[0:9]
        recv_bufs = rest[9:18]
        send_sems, recv_sems = rest[18:]

        d = lax.axis_index("i")
        m = d % 4
        base = d - m
        partners = (
            base + (m ^ 1),
            base + (m ^ 3),
            d ^ 4,
        )
        bits = (
            (m ^ (m >> 1)) & 1,
            m >> 1,
            d >> 2,
        )

        barrier_sem = pltpu.get_barrier_semaphore()
        for ax in range(3):
            pl.semaphore_signal(
                barrier_sem, inc=1,
                device_id=(partners[ax],),
                device_id_type=pl.DeviceIdType.MESH,
            )
        pl.semaphore_wait(barrier_sem, 3)

        acc_ref[...] = jnp.dot(
            x_ref[...], w_ref[...], preferred_element_type=jnp.float32
        )

        def exchange(i, src_ref, dst_ref, ax):
            rdma = pltpu.make_async_remote_copy(
                src_ref=src_ref,
                dst_ref=dst_ref,
                send_sem=send_sems.at[i],
                recv_sem=recv_sems.at[i],
                device_id=(partners[ax],),
                device_id_type=pl.DeviceIdType.MESH,
            )
            rdma.start()
            return rdma

        halves = [r // 2 for _, r in TREES]
        keeps = []
        rdmas = []
        for t, (r0, r) in enumerate(TREES):
            h = halves[t]
            b = bits[AXORDER[t][0]]
            send_start = r0 + (1 - b) * h
            keeps.append(r0 + b * h)
            send_bufs[t * 3][...] = acc_ref[
                pl.ds(send_start, h), :
            ].astype(jnp.bfloat16)
            rdmas.append(exchange(t * 3, send_bufs[t * 3], recv_bufs[t * 3],
                                  AXORDER[t][0]))
        for t in range(3):
            rdmas[t].wait()

        for s in (1, 2):
            rdmas = []
            for t in range(3):
                h = halves[t]
                i = t * 3 + s
                tmp = acc_ref[pl.ds(keeps[t], h), :] + recv_bufs[
                    i - 1
                ][...].astype(jnp.float32)
                acc_ref[pl.ds(keeps[t], h), :] = tmp
                send_bufs[i][...] = tmp.astype(jnp.bfloat16)
                rdmas.append(exchange(i, send_bufs[i], recv_bufs[i],
                                      AXORDER[t][s]))
            for t in range(3):
                rdmas[t].wait()

        for t in range(3):
            h = halves[t]
            z = acc_ref[pl.ds(keeps[t], h), :] + recv_bufs[
                t * 3 + 2
            ][...].astype(jnp.float32)
            ag_ref[pl.ds(keeps[t], h), :] = (z * jax.nn.sigmoid(z)).astype(
                jnp.bfloat16
            )

        rdmas = []
        for t in range(3):
            h = halves[t]
            i = 9 + t
            rdma = pltpu.make_async_remote_copy(
                src_ref=ag_ref.at[pl.ds(keeps[t], h)],
                dst_ref=ag_ref.at[pl.ds(keeps[t], h)],
                send_sem=send_sems.at[i],
                recv_sem=recv_sems.at[i],
                device_id=(partners[AXORDER[t][0]],),
                device_id_type=pl.DeviceIdType.MESH,
            )
            rdma.start()
            rdmas.append(rdma)
        for t in range(3):
            rdmas[t].wait()

        out_ref[...] = ag_ref[...].astype(jnp.float32)

    scratch = [pltpu.VMEM((M, N), jnp.float32), pltpu.VMEM((M, N), jnp.bfloat16)]
    for _, r in TREES:
        scratch += [pltpu.VMEM((r // 2, N), jnp.bfloat16)] * 3
    for _, r in TREES:
        scratch += [pltpu.VMEM((r // 2, N), jnp.bfloat16)] * 3
    scratch += [pltpu.SemaphoreType.DMA((N_EX,)) for _ in range(2)]

    return pl.pallas_call(
        body,
        out_shape=jax.ShapeDtypeStruct((M, N), jnp.float32),
        in_specs=[
            pl.BlockSpec(memory_space=pltpu.VMEM),
            pl.BlockSpec(memory_space=pltpu.VMEM),
        ],
        out_specs=pl.BlockSpec(memory_space=pltpu.VMEM),
        scratch_shapes=scratch,
        compiler_params=pltpu.CompilerParams(collective_id=0),
    )(x, w_mat)


# device time: 26516 ns/iter; 2.7694x vs baseline; 1.2297x over previous
import jax
import jax.numpy as jnp
from jax import lax
from jax.experimental import pallas as pl
from jax.experimental.pallas import tpu as pltpu

N_DEV = 8
M = 1024
N = 1024
C = N // 2

TREES = ((0, 352), (352, 352), (704, 320))
AXORDER = ((0, 1, 2), (1, 2, 0), (2, 0, 1))
N_EX = 24


def kernel(x, w_mat):
    def body(x_ref, w_ref, out_ref, acc_ref, ag_ref, *rest):
        send_bufs = rest[0:9]
        recv_bufs = rest[9:18]
        send_sems, recv_sems = rest[18:]

        d = lax.axis_index("i")
        m = d % 4
        base = d - m
        partners = (
            base + (m ^ 1),
            base + (m ^ 3),
            d ^ 4,
        )
        bits = (
            (m ^ (m >> 1)) & 1,
            m >> 1,
            d >> 2,
        )

        barrier_sem = pltpu.get_barrier_semaphore()
        for ax in range(3):
            pl.semaphore_signal(
                barrier_sem, inc=1,
                device_id=(partners[ax],),
                device_id_type=pl.DeviceIdType.MESH,
            )
        pl.semaphore_wait(barrier_sem, 3)

        acc_ref[...] = jnp.dot(
            x_ref[...], w_ref[...], preferred_element_type=jnp.float32
        )

        halves = [r // 2 for _, r in TREES]
        keeps = [r0 + bits[AXORDER[t][0]] * (r // 2)
                 for t, (r0, r) in enumerate(TREES)]

        def exchange(i, j, src_ref, dst_ref, ax):
            rdma = pltpu.make_async_remote_copy(
                src_ref=src_ref,
                dst_ref=dst_ref,
                send_sem=send_sems.at[i * 2 + j],
                recv_sem=recv_sems.at[i * 2 + j],
                device_id=(partners[ax],),
                device_id_type=pl.DeviceIdType.MESH,
            )
            rdma.start()
            return rdma

        rd = {}
        for j in (0, 1):
            cs = pl.ds(j * C, C)
            for t, (r0, r) in enumerate(TREES):
                h = halves[t]
                b = bits[AXORDER[t][0]]
                send_start = r0 + (1 - b) * h
                i = t * 3
                send_bufs[i][:, cs] = acc_ref[
                    pl.ds(send_start, h), cs
                ].astype(jnp.bfloat16)
                rd[(0, t, j)] = exchange(
                    i, j, send_bufs[i].at[:, cs], recv_bufs[i].at[:, cs],
                    AXORDER[t][0],
                )

        for s in (1, 2):
            for j in (0, 1):
                cs = pl.ds(j * C, C)
                for t in range(3):
                    h = halves[t]
                    i = t * 3 + s
                    rd[(s - 1, t, j)].wait()
                    tmp = acc_ref[pl.ds(keeps[t], h), cs] + recv_bufs[
                        i - 1
                    ][:, cs].astype(jnp.float32)
                    acc_ref[pl.ds(keeps[t], h), cs] = tmp
                    send_bufs[i][:, cs] = tmp.astype(jnp.bfloat16)
                    rd[(s, t, j)] = exchange(
                        i, j, send_bufs[i].at[:, cs], recv_bufs[i].at[:, cs],
                        AXORDER[t][s],
                    )

        for j in (0, 1):
            cs = pl.ds(j * C, C)
            for t in range(3):
                h = halves[t]
                rd[(2, t, j)].wait()
                z = acc_ref[pl.ds(keeps[t], h), cs] + recv_bufs[
                    t * 3 + 2
                ][:, cs].astype(jnp.float32)
                y = z * jax.nn.sigmoid(z)
                out_ref[pl.ds(keeps[t], h), cs] = y
                ag_ref[pl.ds(keeps[t], h), cs] = y.astype(jnp.bfloat16)
                rd[(3, t, j)] = exchange(
                    9 + t, j,
                    ag_ref.at[pl.ds(keeps[t], h), cs],
                    ag_ref.at[pl.ds(keeps[t], h), cs],
                    AXORDER[t][0],
                )

        for j in (0, 1):
            for t in range(3):
                rd[(3, t, j)].wait()
        for t, (r0, r) in enumerate(TREES):
            h = halves[t]
            b = bits[AXORDER[t][0]]
            comp = r0 + (1 - b) * h
            out_ref[pl.ds(comp, h), :] = ag_ref[
                pl.ds(comp, h), :
            ].astype(jnp.float32)

    scratch = [pltpu.VMEM((M, N), jnp.float32), pltpu.VMEM((M, N), jnp.bfloat16)]
    for _, r in TREES:
        scratch += [pltpu.VMEM((r // 2, N), jnp.bfloat16)] * 3
    for _, r in TREES:
        scratch += [pltpu.VMEM((r // 2, N), jnp.bfloat16)] * 3
    scratch += [pltpu.SemaphoreType.DMA((N_EX,)) for _ in range(2)]

    return pl.pallas_call(
        body,
        out_shape=jax.ShapeDtypeStruct((M, N), jnp.float32),
        in_specs=[
            pl.BlockSpec(memory_space=pltpu.VMEM),
            pl.BlockSpec(memory_space=pltpu.VMEM),
        ],
        out_specs=pl.BlockSpec(memory_space=pltpu.VMEM),
        scratch_shapes=scratch,
        compiler_params=pltpu.CompilerParams(collective_id=0),
    )(x, w_mat)


# device time: 25715 ns/iter; 2.8556x vs baseline; 1.0311x over previous
import jax
import jax.numpy as jnp
from jax import lax
from jax.experimental import pallas as pl
from jax.experimental.pallas import tpu as pltpu

N_DEV = 8
M = 1024
N = 1024
NSUB = 4
C = N // NSUB

TREES = ((0, 352), (352, 352), (704, 320))
AXORDER = ((0, 1, 2), (1, 2, 0), (2, 0, 1))
N_EX = 12 * NSUB


def kernel(x, w_mat):
    def body(x_ref, w_ref, out_ref, acc_ref, ag_ref, *rest):
        send_bufs = rest[0:9]
        recv_bufs = rest[9:18]
        send_sems, recv_sems = rest[18:]

        d = lax.axis_index("i")
        m = d % 4
        base = d - m
        partners = (
            base + (m ^ 1),
            base + (m ^ 3),
            d ^ 4,
        )
        bits = (
            (m ^ (m >> 1)) & 1,
            m >> 1,
            d >> 2,
        )

        barrier_sem = pltpu.get_barrier_semaphore()
        for ax in range(3):
            pl.semaphore_signal(
                barrier_sem, inc=1,
                device_id=(partners[ax],),
                device_id_type=pl.DeviceIdType.MESH,
            )
        pl.semaphore_wait(barrier_sem, 3)

        halves = [r // 2 for _, r in TREES]
        keeps = [r0 + bits[AXORDER[t][0]] * (r // 2)
                 for t, (r0, r) in enumerate(TREES)]

        def exchange(i, j, src_ref, dst_ref, ax):
            rdma = pltpu.make_async_remote_copy(
                src_ref=src_ref,
                dst_ref=dst_ref,
                send_sem=send_sems.at[i * NSUB + j],
                recv_sem=recv_sems.at[i * NSUB + j],
                device_id=(partners[ax],),
                device_id_type=pl.DeviceIdType.MESH,
            )
            rdma.start()
            return rdma

        rd = {}
        for j in range(NSUB):
            cs = pl.ds(j * C, C)
            acc_ref[:, cs] = jnp.dot(
                x_ref[...], w_ref[:, cs], preferred_element_type=jnp.float32
            )
            for t, (r0, r) in enumerate(TREES):
                h = halves[t]
                b = bits[AXORDER[t][0]]
                send_start = r0 + (1 - b) * h
                i = t * 3
                send_bufs[i][:, cs] = acc_ref[
                    pl.ds(send_start, h), cs
                ].astype(jnp.bfloat16)
                rd[(0, t, j)] = exchange(
                    i, j, send_bufs[i].at[:, cs], recv_bufs[i].at[:, cs],
                    AXORDER[t][0],
                )

        for s in (1, 2):
            for j in range(NSUB):
                cs = pl.ds(j * C, C)
                for t in range(3):
                    h = halves[t]
                    i = t * 3 + s
                    rd[(s - 1, t, j)].wait()
                    tmp = acc_ref[pl.ds(keeps[t], h), cs] + recv_bufs[
                        i - 1
                    ][:, cs].astype(jnp.float32)
                    acc_ref[pl.ds(keeps[t], h), cs] = tmp
                    send_bufs[i][:, cs] = tmp.astype(jnp.bfloat16)
                    rd[(s, t, j)] = exchange(
                        i, j, send_bufs[i].at[:, cs], recv_bufs[i].at[:, cs],
                        AXORDER[t][s],
                    )

        for j in range(NSUB):
            cs = pl.ds(j * C, C)
            for t in range(3):
                h = halves[t]
                rd[(2, t, j)].wait()
                z = acc_ref[pl.ds(keeps[t], h), cs] + recv_bufs[
                    t * 3 + 2
                ][:, cs].astype(jnp.float32)
                y = z * jax.nn.sigmoid(z)
                out_ref[pl.ds(keeps[t], h), cs] = y
                ag_ref[pl.ds(keeps[t], h), cs] = y.astype(jnp.bfloat16)
                rd[(3, t, j)] = exchange(
                    9 + t, j,
                    ag_ref.at[pl.ds(keeps[t], h), cs],
                    ag_ref.at[pl.ds(keeps[t], h), cs],
                    AXORDER[t][0],
                )

        for j in range(NSUB):
            cs = pl.ds(j * C, C)
            for t, (r0, r) in enumerate(TREES):
                h = halves[t]
                b = bits[AXORDER[t][0]]
                comp = r0 + (1 - b) * h
                rd[(3, t, j)].wait()
                out_ref[pl.ds(comp, h), cs] = ag_ref[
                    pl.ds(comp, h), cs
                ].astype(jnp.float32)

    scratch = [pltpu.VMEM((M, N), jnp.float32), pltpu.VMEM((M, N), jnp.bfloat16)]
    for _, r in TREES:
        scratch += [pltpu.VMEM((r // 2, N), jnp.bfloat16)] * 3
    for _, r in TREES:
        scratch += [pltpu.VMEM((r // 2, N), jnp.bfloat16)] * 3
    scratch += [pltpu.SemaphoreType.DMA((N_EX,)) for _ in range(2)]

    return pl.pallas_call(
        body,
        out_shape=jax.ShapeDtypeStruct((M, N), jnp.float32),
        in_specs=[
            pl.BlockSpec(memory_space=pltpu.VMEM),
            pl.BlockSpec(memory_space=pltpu.VMEM),
        ],
        out_specs=pl.BlockSpec(memory_space=pltpu.VMEM),
        scratch_shapes=scratch,
        compiler_params=pltpu.CompilerParams(collective_id=0),
    )(x, w_mat)
